# baseline (speedup 1.0000x reference)
import jax
import jax.numpy as jnp
from jax import lax
from jax.experimental import pallas as pl
from jax.experimental.pallas import tpu as pltpu
from jax.experimental.pallas import tpu_sc as plsc

N_NODES = 10000
N_EDGES = 320000
D = 128
DH = 64
NC = 2
NS = 16
L = 16
CHUNK = 200
NB = 20
NG = 4
EPT = N_EDGES // NS
NCH = EPT // CHUNK
NSUP = NCH // NB
ROWS_PT = 624
REM_ROWS = N_NODES - NS * ROWS_PT


def _zero_buf(buf):
    zeros = jnp.zeros((L,), jnp.float32)

    def zb(e, c):
        for d in range(DH // L):
            buf[e, pl.ds(d * L, L)] = zeros
        return c

    lax.fori_loop(0, CHUNK, zb, 0)


def _zero_acc(acc, zbuf, sid):
    r0 = pl.multiple_of(sid * ROWS_PT, 8)
    full = ROWS_PT // CHUNK
    rem = ROWS_PT - full * CHUNK
    for j in range(full):
        pltpu.sync_copy(zbuf, acc.at[pl.ds(r0 + j * CHUNK, CHUNK)])
    if rem:
        pltpu.sync_copy(zbuf.at[pl.ds(0, rem)],
                        acc.at[pl.ds(r0 + full * CHUNK, rem)])

    @pl.when(sid == NS - 1)
    def _():
        pltpu.sync_copy(zbuf.at[pl.ds(0, REM_ROWS)],
                        acc.at[pl.ds(NS * ROWS_PT, REM_ROWS)])


HC = CHUNK // 2


def _fire_gather(src, colv, gbuf, gsem, i):
    pltpu.async_copy(src.at[colv.at[2 * i]], gbuf.at[pl.ds(0, HC)], gsem)
    pltpu.async_copy(src.at[colv.at[2 * i + 1]], gbuf.at[pl.ds(HC, HC)], gsem)


def _wait_gather(src, colv, gbuf, gsem, i):
    pltpu.make_async_copy(src.at[colv.at[2 * i]],
                          gbuf.at[pl.ds(0, HC)], gsem).wait()
    pltpu.make_async_copy(src.at[colv.at[2 * i + 1]],
                          gbuf.at[pl.ds(HC, HC)], gsem).wait()


def _hop(src, dst, sid, col_r, row_r, w_hbm,
         colv, rowv, wv, gbufs, sbufs, gsems, ssems):
    def super_body(j, c0):
        pltpu.sync_copy(col_r.at[sid, j], colv)
        pltpu.sync_copy(row_r.at[sid, j], rowv)
        wbase = pl.multiple_of(sid * EPT + j * (NB * CHUNK), 8)
        pltpu.sync_copy(w_hbm.at[pl.ds(wbase, NB * CHUNK)], wv)

        for b in range(NG - 1):
            _fire_gather(src, colv, gbufs[b], gsems[b], b)

        def quad_body(k, c):
            for b in range(NG):
                i = k * NG + b
                sb = b % 2
                _wait_gather(src, colv, gbufs[b], gsems[b], i)

                @pl.when(i + NG - 1 < NB)
                def _():
                    _fire_gather(src, colv, gbufs[(b + NG - 1) % NG],
                                 gsems[(b + NG - 1) % NG], i + NG - 1)

                @pl.when(i >= 2)
                def _():
                    pltpu.make_async_copy(
                        sbufs[sb], dst.at[rowv.at[i - 2]], ssems[sb]).wait()

                @plsc.parallel_loop(0, CHUNK, unroll=8)
                def scale(e):
                    wb = plsc.load_gather(
                        wv, [jnp.full((L,), i * CHUNK + e, jnp.int32)])
                    for d in range(DH // L):
                        sl = pl.ds(d * L, L)
                        sbufs[sb][e, sl] = gbufs[b][e, sl] * wb

                pltpu.async_copy(sbufs[sb], dst.at[rowv.at[i]], ssems[sb],
                                 add=True)

            return c

        lax.fori_loop(0, NB // NG, quad_body, 0)

        for i in (NB - 2, NB - 1):
            pltpu.make_async_copy(sbufs[i % 2], dst.at[rowv.at[i]],
                                  ssems[i % 2]).wait()
        return c0

    lax.fori_loop(0, NSUP, super_body, 0)


def _body(x0, x1, col_r, row_r, w_hbm, o0, o1, h0, h1,
          acc, colv, rowv, wv, gbuf0, gbuf1, gbuf2, gbuf3, sbuf0, sbuf1,
          gsem0, gsem1, gsem2, gsem3, ssem0, ssem1):
    cid = lax.axis_index("c")
    sid = lax.axis_index("s")
    gbufs = (gbuf0, gbuf1, gbuf2, gbuf3)
    sbufs = (sbuf0, sbuf1)
    gsems = (gsem0, gsem1, gsem2, gsem3)
    ssems = (ssem0, ssem1)

    r0 = pl.multiple_of(sid * ROWS_PT, 8)
    tail = NS * ROWS_PT

    def _writeback_from(a, o):
        pltpu.sync_copy(a.at[pl.ds(r0, ROWS_PT)], o.at[pl.ds(r0, ROWS_PT)])

        @pl.when(sid == NS - 1)
        def _():
            pltpu.sync_copy(a.at[pl.ds(tail, REM_ROWS)],
                            o.at[pl.ds(tail, REM_ROWS)])

    _zero_buf(sbuf0)
    _zero_acc(acc, sbuf0, sid)
    plsc.subcore_barrier()

    @pl.when(cid == 0)
    def _():
        _hop(x0, acc, sid, col_r, row_r, w_hbm,
             colv, rowv, wv, gbufs, sbufs, gsems, ssems)

    @pl.when(cid == 1)
    def _():
        _hop(x1, acc, sid, col_r, row_r, w_hbm,
             colv, rowv, wv, gbufs, sbufs, gsems, ssems)

    plsc.subcore_barrier()

    @pl.when(cid == 0)
    def _():
        _writeback_from(acc, h0)

    @pl.when(cid == 1)
    def _():
        _writeback_from(acc, h1)

    plsc.subcore_barrier()
    _zero_buf(sbuf0)
    _zero_acc(acc, sbuf0, sid)
    plsc.subcore_barrier()

    @pl.when(cid == 0)
    def _():
        _hop(h0, acc, sid, col_r, row_r, w_hbm,
             colv, rowv, wv, gbufs, sbufs, gsems, ssems)

    @pl.when(cid == 1)
    def _():
        _hop(h1, acc, sid, col_r, row_r, w_hbm,
             colv, rowv, wv, gbufs, sbufs, gsems, ssems)

    plsc.subcore_barrier()

    @pl.when(cid == 0)
    def _():
        _writeback_from(acc, o0)

    @pl.when(cid == 1)
    def _():
        _writeback_from(acc, o1)


def kernel(x, edge_index, edge_values):
    x0 = x[:, :DH]
    x1 = x[:, DH:]
    row_r = edge_index[0].reshape(NS, NSUP, NB, CHUNK)
    col_r = edge_index[1].reshape(NS, NSUP, NB * 2, CHUNK // 2)
    w_r = edge_values

    f = pl.kernel(
        _body,
        out_type=(jax.ShapeDtypeStruct((N_NODES, DH), jnp.float32),
                  jax.ShapeDtypeStruct((N_NODES, DH), jnp.float32),
                  jax.ShapeDtypeStruct((N_NODES, DH), jnp.float32),
                  jax.ShapeDtypeStruct((N_NODES, DH), jnp.float32)),
        mesh=plsc.VectorSubcoreMesh(core_axis_name="c", subcore_axis_name="s",
                                    num_cores=NC, num_subcores=NS),
        compiler_params=pltpu.CompilerParams(needs_layout_passes=False,
                                             use_tc_tiling_on_sc=False),
        scratch_types=[
            pltpu.VMEM_SHARED((N_NODES, DH), jnp.float32),
            pltpu.VMEM((NB * 2, CHUNK // 2), jnp.int32),
            pltpu.VMEM((NB, CHUNK), jnp.int32),
            pltpu.VMEM((NB * CHUNK,), jnp.float32),
            pltpu.VMEM((CHUNK, DH), jnp.float32),
            pltpu.VMEM((CHUNK, DH), jnp.float32),
            pltpu.VMEM((CHUNK, DH), jnp.float32),
            pltpu.VMEM((CHUNK, DH), jnp.float32),
            pltpu.VMEM((CHUNK, DH), jnp.float32),
            pltpu.VMEM((CHUNK, DH), jnp.float32),
            pltpu.SemaphoreType.DMA,
            pltpu.SemaphoreType.DMA,
            pltpu.SemaphoreType.DMA,
            pltpu.SemaphoreType.DMA,
            pltpu.SemaphoreType.DMA,
            pltpu.SemaphoreType.DMA,
        ],
    )
    o0, o1, _, _ = f(x0, x1, col_r, row_r, w_r)
    return jnp.concatenate([o0, o1], axis=1)


# speedup vs baseline: 1.0796x; 1.0796x over previous
import jax
import jax.numpy as jnp
from jax import lax
from jax.experimental import pallas as pl
from jax.experimental.pallas import tpu as pltpu
from jax.experimental.pallas import tpu_sc as plsc

N_NODES = 10000
N_EDGES = 320000
D = 128
DH = 64
NC = 2
NS = 16
L = 16
CHUNK = 200
NB = 20
NG = 4
EPT = N_EDGES // NS
NCH = EPT // CHUNK
NSUP = NCH // NB
ROWS_PT = 624
REM_ROWS = N_NODES - NS * ROWS_PT


def _zero_buf(buf):
    zeros = jnp.zeros((L,), jnp.float32)

    def zb(e, c):
        for d in range(DH // L):
            buf[e, pl.ds(d * L, L)] = zeros
        return c

    lax.fori_loop(0, CHUNK, zb, 0)


def _zero_acc(acc, zbuf, sid):
    r0 = pl.multiple_of(sid * ROWS_PT, 8)
    full = ROWS_PT // CHUNK
    rem = ROWS_PT - full * CHUNK
    for j in range(full):
        pltpu.sync_copy(zbuf, acc.at[pl.ds(r0 + j * CHUNK, CHUNK)])
    if rem:
        pltpu.sync_copy(zbuf.at[pl.ds(0, rem)],
                        acc.at[pl.ds(r0 + full * CHUNK, rem)])

    @pl.when(sid == NS - 1)
    def _():
        pltpu.sync_copy(zbuf.at[pl.ds(0, REM_ROWS)],
                        acc.at[pl.ds(NS * ROWS_PT, REM_ROWS)])


def _hop(src, dst, sid, col_r, row_r, w_hbm,
         colv, rowv, wv, gbufs, sbufs, gsems, ssems, isem):
    def super_body(j, c0):
        wbase = pl.multiple_of(sid * EPT + j * (NB * CHUNK), 8)
        c1 = pltpu.async_copy(col_r.at[sid, j], colv, isem)
        c2 = pltpu.async_copy(row_r.at[sid, j], rowv, isem)
        c3 = pltpu.async_copy(w_hbm.at[pl.ds(wbase, NB * CHUNK)], wv, isem)
        c1.wait()
        c2.wait()
        c3.wait()

        for b in range(NG - 1):
            pltpu.async_copy(src.at[colv.at[b]], gbufs[b], gsems[b])

        def quad_body(k, c):
            for b in range(NG):
                i = k * NG + b
                sb = b % 2
                pltpu.make_async_copy(src.at[colv.at[i]],
                                      gbufs[b], gsems[b]).wait()

                @pl.when(i + NG - 1 < NB)
                def _():
                    pltpu.async_copy(src.at[colv.at[i + NG - 1]],
                                     gbufs[(b + NG - 1) % NG],
                                     gsems[(b + NG - 1) % NG])

                @pl.when(i >= 2)
                def _():
                    pltpu.make_async_copy(
                        sbufs[sb], dst.at[rowv.at[i - 2]], ssems[sb]).wait()

                @plsc.parallel_loop(0, CHUNK, unroll=8)
                def scale(e):
                    wb = plsc.load_gather(
                        wv, [jnp.full((L,), i * CHUNK + e, jnp.int32)])
                    for d in range(DH // L):
                        sl = pl.ds(d * L, L)
                        sbufs[sb][e, sl] = gbufs[b][e, sl] * wb

                pltpu.async_copy(sbufs[sb], dst.at[rowv.at[i]], ssems[sb],
                                 add=True)

            return c

        lax.fori_loop(0, NB // NG, quad_body, 0)

        for i in (NB - 2, NB - 1):
            pltpu.make_async_copy(sbufs[i % 2], dst.at[rowv.at[i]],
                                  ssems[i % 2]).wait()
        return c0

    lax.fori_loop(0, NSUP, super_body, 0)


def _body(x0, x1, col_r, row_r, w_hbm, o0, o1, h0, h1,
          acc, colv, rowv, wv, gbuf0, gbuf1, gbuf2, gbuf3, sbuf0, sbuf1,
          gsem0, gsem1, gsem2, gsem3, ssem0, ssem1, isem):
    cid = lax.axis_index("c")
    sid = lax.axis_index("s")
    gbufs = (gbuf0, gbuf1, gbuf2, gbuf3)
    sbufs = (sbuf0, sbuf1)
    gsems = (gsem0, gsem1, gsem2, gsem3)
    ssems = (ssem0, ssem1)

    r0 = pl.multiple_of(sid * ROWS_PT, 8)
    tail = NS * ROWS_PT

    def _writeback_from(a, o):
        pltpu.sync_copy(a.at[pl.ds(r0, ROWS_PT)], o.at[pl.ds(r0, ROWS_PT)])

        @pl.when(sid == NS - 1)
        def _():
            pltpu.sync_copy(a.at[pl.ds(tail, REM_ROWS)],
                            o.at[pl.ds(tail, REM_ROWS)])

    _zero_buf(sbuf0)
    _zero_acc(acc, sbuf0, sid)
    plsc.subcore_barrier()

    @pl.when(cid == 0)
    def _():
        _hop(x0, acc, sid, col_r, row_r, w_hbm,
             colv, rowv, wv, gbufs, sbufs, gsems, ssems, isem)

    @pl.when(cid == 1)
    def _():
        _hop(x1, acc, sid, col_r, row_r, w_hbm,
             colv, rowv, wv, gbufs, sbufs, gsems, ssems, isem)

    plsc.subcore_barrier()

    @pl.when(cid == 0)
    def _():
        _writeback_from(acc, h0)

    @pl.when(cid == 1)
    def _():
        _writeback_from(acc, h1)

    plsc.subcore_barrier()
    _zero_buf(sbuf0)
    _zero_acc(acc, sbuf0, sid)
    plsc.subcore_barrier()

    @pl.when(cid == 0)
    def _():
        _hop(h0, acc, sid, col_r, row_r, w_hbm,
             colv, rowv, wv, gbufs, sbufs, gsems, ssems, isem)

    @pl.when(cid == 1)
    def _():
        _hop(h1, acc, sid, col_r, row_r, w_hbm,
             colv, rowv, wv, gbufs, sbufs, gsems, ssems, isem)

    plsc.subcore_barrier()

    @pl.when(cid == 0)
    def _():
        _writeback_from(acc, o0)

    @pl.when(cid == 1)
    def _():
        _writeback_from(acc, o1)


def kernel(x, edge_index, edge_values):
    x0 = x[:, :DH]
    x1 = x[:, DH:]
    row_r = edge_index[0].reshape(NS, NSUP, NB, CHUNK)
    col_r = edge_index[1].reshape(NS, NSUP, NB, CHUNK)
    w_r = edge_values

    f = pl.kernel(
        _body,
        out_type=(jax.ShapeDtypeStruct((N_NODES, DH), jnp.float32),
                  jax.ShapeDtypeStruct((N_NODES, DH), jnp.float32),
                  jax.ShapeDtypeStruct((N_NODES, DH), jnp.float32),
                  jax.ShapeDtypeStruct((N_NODES, DH), jnp.float32)),
        mesh=plsc.VectorSubcoreMesh(core_axis_name="c", subcore_axis_name="s",
                                    num_cores=NC, num_subcores=NS),
        compiler_params=pltpu.CompilerParams(needs_layout_passes=False,
                                             use_tc_tiling_on_sc=False),
        scratch_types=[
            pltpu.VMEM_SHARED((N_NODES, DH), jnp.float32),
            pltpu.VMEM((NB, CHUNK), jnp.int32),
            pltpu.VMEM((NB, CHUNK), jnp.int32),
            pltpu.VMEM((NB * CHUNK,), jnp.float32),
            pltpu.VMEM((CHUNK, DH), jnp.float32),
            pltpu.VMEM((CHUNK, DH), jnp.float32),
            pltpu.VMEM((CHUNK, DH), jnp.float32),
            pltpu.VMEM((CHUNK, DH), jnp.float32),
            pltpu.VMEM((CHUNK, DH), jnp.float32),
            pltpu.VMEM((CHUNK, DH), jnp.float32),
            pltpu.SemaphoreType.DMA,
            pltpu.SemaphoreType.DMA,
            pltpu.SemaphoreType.DMA,
            pltpu.SemaphoreType.DMA,
            pltpu.SemaphoreType.DMA,
            pltpu.SemaphoreType.DMA,
            pltpu.SemaphoreType.DMA,
        ],
    )
    o0, o1, _, _ = f(x0, x1, col_r, row_r, w_r)
    return jnp.concatenate([o0, o1], axis=1)


# speedup vs baseline: 1.2385x; 1.1472x over previous
import jax
import jax.numpy as jnp
from jax import lax
from jax.experimental import pallas as pl
from jax.experimental.pallas import tpu as pltpu
from jax.experimental.pallas import tpu_sc as plsc

N_NODES = 10000
N_EDGES = 320000
D = 128
DH = 64
NC = 2
NS = 16
L = 16
CHUNK = 200
NB = 20
NG = 4
EPT = N_EDGES // NS
NCH = EPT // CHUNK
NSUP = NCH // NB
ROWS_PT = 624
REM_ROWS = N_NODES - NS * ROWS_PT
PACKED = plsc.PackFormat.INTERLEAVED


def _zero_buf(buf):
    zeros = jnp.zeros((L,), jnp.float32)

    def zb(e, c):
        for d in range(DH // L):
            buf[e, pl.ds(d * L, L)] = zeros
        return c

    lax.fori_loop(0, CHUNK, zb, 0)


def _zero_acc(acc, zbuf, sid):
    r0 = pl.multiple_of(sid * ROWS_PT, 8)
    full = ROWS_PT // CHUNK
    rem = ROWS_PT - full * CHUNK
    for j in range(full):
        pltpu.sync_copy(zbuf, acc.at[pl.ds(r0 + j * CHUNK, CHUNK)])
    if rem:
        pltpu.sync_copy(zbuf.at[pl.ds(0, rem)],
                        acc.at[pl.ds(r0 + full * CHUNK, rem)])

    @pl.when(sid == NS - 1)
    def _():
        pltpu.sync_copy(zbuf.at[pl.ds(0, REM_ROWS)],
                        acc.at[pl.ds(NS * ROWS_PT, REM_ROWS)])


def _conv_rows(fbuf, bbuf, n):
    @plsc.parallel_loop(0, n, unroll=8)
    def cv(e):
        for d2 in range(DH // 32):
            a = fbuf[e, pl.ds(d2 * 32, L)]
            b = fbuf[e, pl.ds(d2 * 32 + L, L)]
            bbuf[e, pl.ds(d2 * 32, 32)] = plsc.pack(a, b, format=PACKED)


def _convert(src, dst, sid, fbuf, bbuf):
    r0 = pl.multiple_of(sid * ROWS_PT, 8)
    full = ROWS_PT // CHUNK
    rem = ROWS_PT - full * CHUNK
    pieces = [(r0 + j * CHUNK, CHUNK) for j in range(full)]
    if rem:
        pieces.append((r0 + full * CHUNK, rem))
    for base, n in pieces:
        pltpu.sync_copy(src.at[pl.ds(base, n)], fbuf.at[pl.ds(0, n)])
        _conv_rows(fbuf, bbuf, n)
        pltpu.sync_copy(bbuf.at[pl.ds(0, n)], dst.at[pl.ds(base, n)])

    @pl.when(sid == NS - 1)
    def _():
        tail = NS * ROWS_PT
        pltpu.sync_copy(src.at[pl.ds(tail, REM_ROWS)],
                        fbuf.at[pl.ds(0, REM_ROWS)])
        _conv_rows(fbuf, bbuf, REM_ROWS)
        pltpu.sync_copy(bbuf.at[pl.ds(0, REM_ROWS)],
                        dst.at[pl.ds(tail, REM_ROWS)])


def _hop(src, dst, sid, col_r, row_r, w_hbm,
         colv, rowv, wv, gbufs, sbufs, gsems, ssems, isem):
    def super_body(j, c0):
        wbase = pl.multiple_of(sid * EPT + j * (NB * CHUNK), 8)
        c1 = pltpu.async_copy(col_r.at[sid, j], colv, isem)
        c2 = pltpu.async_copy(row_r.at[sid, j], rowv, isem)
        c3 = pltpu.async_copy(w_hbm.at[pl.ds(wbase, NB * CHUNK)], wv, isem)
        c1.wait()
        c2.wait()
        c3.wait()

        for b in range(NG - 1):
            pltpu.async_copy(src.at[colv.at[b]], gbufs[b], gsems[b])

        def quad_body(k, c):
            for b in range(NG):
                i = k * NG + b
                sb = b % 2
                pltpu.make_async_copy(src.at[colv.at[i]],
                                      gbufs[b], gsems[b]).wait()

                @pl.when(i + NG - 1 < NB)
                def _():
                    pltpu.async_copy(src.at[colv.at[i + NG - 1]],
                                     gbufs[(b + NG - 1) % NG],
                                     gsems[(b + NG - 1) % NG])

                @pl.when(i >= 2)
                def _():
                    pltpu.make_async_copy(
                        sbufs[sb], dst.at[rowv.at[i - 2]], ssems[sb]).wait()

                @plsc.parallel_loop(0, CHUNK, unroll=8)
                def scale(e):
                    wb = plsc.load_gather(
                        wv, [jnp.full((L,), i * CHUNK + e, jnp.int32)])
                    for d2 in range(DH // 32):
                        v = gbufs[b][e, pl.ds(d2 * 32, 32)]
                        a, bb = plsc.unpack(v, format=PACKED)
                        sbufs[sb][e, pl.ds(d2 * 32, L)] = a * wb
                        sbufs[sb][e, pl.ds(d2 * 32 + L, L)] = bb * wb

                pltpu.async_copy(sbufs[sb], dst.at[rowv.at[i]], ssems[sb],
                                 add=True)

            return c

        lax.fori_loop(0, NB // NG, quad_body, 0)

        for i in (NB - 2, NB - 1):
            pltpu.make_async_copy(sbufs[i % 2], dst.at[rowv.at[i]],
                                  ssems[i % 2]).wait()
        return c0

    lax.fori_loop(0, NSUP, super_body, 0)


def _body(x0, x1, col_r, row_r, w_hbm, o0, o1, xb0, xb1, h0, h1,
          acc, colv, rowv, wv, gbuf0, gbuf1, gbuf2, gbuf3, sbuf0, sbuf1,
          gsem0, gsem1, gsem2, gsem3, ssem0, ssem1, isem):
    cid = lax.axis_index("c")
    sid = lax.axis_index("s")
    gbufs = (gbuf0, gbuf1, gbuf2, gbuf3)
    sbufs = (sbuf0, sbuf1)
    gsems = (gsem0, gsem1, gsem2, gsem3)
    ssems = (ssem0, ssem1)

    r0 = pl.multiple_of(sid * ROWS_PT, 8)
    tail = NS * ROWS_PT

    def _writeback_from(a, o):
        pltpu.sync_copy(a.at[pl.ds(r0, ROWS_PT)], o.at[pl.ds(r0, ROWS_PT)])

        @pl.when(sid == NS - 1)
        def _():
            pltpu.sync_copy(a.at[pl.ds(tail, REM_ROWS)],
                            o.at[pl.ds(tail, REM_ROWS)])

    _zero_buf(sbuf0)
    _zero_acc(acc, sbuf0, sid)

    @pl.when(cid == 0)
    def _():
        _convert(x0, xb0, sid, sbuf1, gbuf0)

    @pl.when(cid == 1)
    def _():
        _convert(x1, xb1, sid, sbuf1, gbuf0)

    plsc.subcore_barrier()

    @pl.when(cid == 0)
    def _():
        _hop(xb0, acc, sid, col_r, row_r, w_hbm,
             colv, rowv, wv, gbufs, sbufs, gsems, ssems, isem)

    @pl.when(cid == 1)
    def _():
        _hop(xb1, acc, sid, col_r, row_r, w_hbm,
             colv, rowv, wv, gbufs, sbufs, gsems, ssems, isem)

    plsc.subcore_barrier()

    @pl.when(cid == 0)
    def _():
        _convert(acc, h0, sid, sbuf1, gbuf0)

    @pl.when(cid == 1)
    def _():
        _convert(acc, h1, sid, sbuf1, gbuf0)

    plsc.subcore_barrier()
    _zero_buf(sbuf0)
    _zero_acc(acc, sbuf0, sid)
    plsc.subcore_barrier()

    @pl.when(cid == 0)
    def _():
        _hop(h0, acc, sid, col_r, row_r, w_hbm,
             colv, rowv, wv, gbufs, sbufs, gsems, ssems, isem)

    @pl.when(cid == 1)
    def _():
        _hop(h1, acc, sid, col_r, row_r, w_hbm,
             colv, rowv, wv, gbufs, sbufs, gsems, ssems, isem)

    plsc.subcore_barrier()

    @pl.when(cid == 0)
    def _():
        _writeback_from(acc, o0)

    @pl.when(cid == 1)
    def _():
        _writeback_from(acc, o1)


def kernel(x, edge_index, edge_values):
    x0 = x[:, :DH]
    x1 = x[:, DH:]
    row_r = edge_index[0].reshape(NS, NSUP, NB, CHUNK)
    col_r = edge_index[1].reshape(NS, NSUP, NB, CHUNK)
    w_r = edge_values

    f = pl.kernel(
        _body,
        out_type=(jax.ShapeDtypeStruct((N_NODES, DH), jnp.float32),
                  jax.ShapeDtypeStruct((N_NODES, DH), jnp.float32),
                  jax.ShapeDtypeStruct((N_NODES, DH), jnp.bfloat16),
                  jax.ShapeDtypeStruct((N_NODES, DH), jnp.bfloat16),
                  jax.ShapeDtypeStruct((N_NODES, DH), jnp.bfloat16),
                  jax.ShapeDtypeStruct((N_NODES, DH), jnp.bfloat16)),
        mesh=plsc.VectorSubcoreMesh(core_axis_name="c", subcore_axis_name="s",
                                    num_cores=NC, num_subcores=NS),
        compiler_params=pltpu.CompilerParams(needs_layout_passes=False,
                                             use_tc_tiling_on_sc=False),
        scratch_types=[
            pltpu.VMEM_SHARED((N_NODES, DH), jnp.float32),
            pltpu.VMEM((NB, CHUNK), jnp.int32),
            pltpu.VMEM((NB, CHUNK), jnp.int32),
            pltpu.VMEM((NB * CHUNK,), jnp.float32),
            pltpu.VMEM((CHUNK, DH), jnp.bfloat16),
            pltpu.VMEM((CHUNK, DH), jnp.bfloat16),
            pltpu.VMEM((CHUNK, DH), jnp.bfloat16),
            pltpu.VMEM((CHUNK, DH), jnp.bfloat16),
            pltpu.VMEM((CHUNK, DH), jnp.float32),
            pltpu.VMEM((CHUNK, DH), jnp.float32),
            pltpu.SemaphoreType.DMA,
            pltpu.SemaphoreType.DMA,
            pltpu.SemaphoreType.DMA,
            pltpu.SemaphoreType.DMA,
            pltpu.SemaphoreType.DMA,
            pltpu.SemaphoreType.DMA,
            pltpu.SemaphoreType.DMA,
        ],
    )
    o0, o1, _, _, _, _ = f(x0, x1, col_r, row_r, w_r)
    return jnp.concatenate([o0, o1], axis=1)
